# baseline (speedup 1.0000x reference)
import functools

import jax
import jax.numpy as jnp
from jax import lax
from jax.experimental import pallas as pl
from jax.experimental.pallas import tpu as pltpu
from jax.experimental.pallas import tpu_sc as plsc

NUM_CORES = 2
NUM_SUBCORES = 16
NUM_WORKERS = NUM_CORES * NUM_SUBCORES

BATCH = 16384
DIM = 64
LANES = 16

B_PER_W = BATCH // NUM_WORKERS
CHUNK = 128
NCHUNK = B_PER_W // CHUNK
NTILE = BATCH // CHUNK
GROUPS = DIM // LANES


def _pair_body(i0, i1, t0, t1, o0, o1, idx_v, rows_v, t_v, gsem, ssem):
    wid = lax.axis_index("s") * NUM_CORES + lax.axis_index("c")
    base = wid * B_PER_W

    pltpu.sync_copy(i0.at[pl.ds(base, B_PER_W)], idx_v.at[0])
    pltpu.sync_copy(i1.at[pl.ds(base, B_PER_W)], idx_v.at[1])

    tables = (t0, t1)
    outs = (o0, o1)
    total = 2 * NCHUNK

    lane = lax.iota(jnp.int32, LANES)
    g_idx = [2 * j + (lane >> 3) for j in range(GROUPS)]
    r_idx = lane & 7
    zeros = lane * 0

    def start_gather(c):
        l, cc = divmod(c, NCHUNK)
        return pltpu.async_copy(
            tables[l].at[idx_v.at[l, pl.ds(cc * CHUNK, CHUNK)]],
            rows_v.at[c % 2], gsem.at[c % 2])

    def transpose(c):
        rows = rows_v.at[c % 2]
        t3 = t_v.at[c % 2]

        def step(k, _):
            kv = zeros + k
            for j in range(GROUPS):
                v = rows[k, pl.ds(16 * j, LANES)]
                plsc.store_scatter(t3, [g_idx[j], r_idx, kv], v)
            return 0

        lax.fori_loop(0, CHUNK, step, 0, unroll=8)

    def start_store(c):
        l, cc = divmod(c, NCHUNK)
        return pltpu.async_copy(
            t_v.at[c % 2],
            outs[l].at[:, wid * NCHUNK + cc],
            ssem.at[c % 2])

    g = [None] * total
    s = [None] * total
    g[0] = start_gather(0)
    for c in range(total):
        if c + 1 < total:
            g[c + 1] = start_gather(c + 1)
        g[c].wait()
        if c >= 2:
            s[c - 2].wait()
        transpose(c)
        s[c] = start_store(c)
    s[total - 2].wait()
    s[total - 1].wait()


@jax.jit
def _pair(i0, i1, t0, t1):
    mesh = plsc.VectorSubcoreMesh(core_axis_name="c", subcore_axis_name="s")
    out = jax.ShapeDtypeStruct((GROUPS * 2, NTILE, 8, CHUNK), jnp.float32)
    return pl.kernel(
        _pair_body,
        out_type=(out, out),
        mesh=mesh,
        compiler_params=pltpu.CompilerParams(use_tc_tiling_on_sc=False,
                                             needs_layout_passes=False),
        scratch_types=[
            pltpu.VMEM((2, B_PER_W), jnp.int32),
            pltpu.VMEM((2, CHUNK, DIM), jnp.float32),
            pltpu.VMEM((2, GROUPS * 2, 8, CHUNK), jnp.float32),
            pltpu.SemaphoreType.DMA((2,)),
            pltpu.SemaphoreType.DMA((2,)),
        ],
    )(i0, i1, t0, t1)


def _untranspose(o4):
    return o4.transpose(1, 3, 0, 2).reshape(BATCH, DIM)


def kernel(head, relation, tail, timestamp,
           entity_table, relation_table, timestamp_table):
    rel_o, ts_o = _pair(relation.astype(jnp.int32),
                        timestamp.astype(jnp.int32),
                        relation_table, timestamp_table)
    head_o, tail_o = _pair(head.astype(jnp.int32),
                           tail.astype(jnp.int32),
                           entity_table, entity_table)
    return (_untranspose(head_o), _untranspose(rel_o),
            _untranspose(tail_o), _untranspose(ts_o))


# speedup vs baseline: 1.0573x; 1.0573x over previous
import functools

import jax
import jax.numpy as jnp
from jax import lax
from jax.experimental import pallas as pl
from jax.experimental.pallas import tpu as pltpu
from jax.experimental.pallas import tpu_sc as plsc

NUM_CORES = 2
NUM_SUBCORES = 16
NUM_WORKERS = NUM_CORES * NUM_SUBCORES

BATCH = 16384
DIM = 64
LANES = 16

B_PER_W = BATCH // NUM_WORKERS
CHUNK = 128
NCHUNK = B_PER_W // CHUNK
NTILE = BATCH // CHUNK
GROUPS = DIM // LANES


def _pair_body(i0, i1, t0, t1, o0, o1, idx_v, rows_v, t_v, gsem, ssem):
    wid = lax.axis_index("s") * NUM_CORES + lax.axis_index("c")
    base = wid * B_PER_W

    pltpu.sync_copy(i0.at[pl.ds(base, B_PER_W)], idx_v.at[0])
    pltpu.sync_copy(i1.at[pl.ds(base, B_PER_W)], idx_v.at[1])

    tables = (t0, t1)
    outs = (o0, o1)
    total = 2 * NCHUNK

    lane = lax.iota(jnp.int32, LANES)
    g_idx = [2 * j + (lane >> 3) for j in range(GROUPS)]
    r_idx = lane & 7
    zeros = lane * 0

    def start_gather(c):
        l, cc = divmod(c, NCHUNK)
        return pltpu.async_copy(
            tables[l].at[idx_v.at[l, pl.ds(cc * CHUNK, CHUNK)]],
            rows_v.at[c % 2], gsem.at[c % 2])

    def transpose(c):
        rows = rows_v.at[c % 2]
        t3 = t_v.at[c % 2]

        def step(k, _):
            kv = zeros + k
            for j in range(GROUPS):
                v = rows[k, pl.ds(16 * j, LANES)]
                plsc.store_scatter(t3, [g_idx[j], r_idx, kv], v)
            return 0

        lax.fori_loop(0, CHUNK, step, 0, unroll=2)

    def start_store(c):
        l, cc = divmod(c, NCHUNK)
        return pltpu.async_copy(
            t_v.at[c % 2],
            outs[l].at[:, wid * NCHUNK + cc],
            ssem.at[c % 2])

    g = [None] * total
    s = [None] * total
    g[0] = start_gather(0)
    for c in range(total):
        if c + 1 < total:
            g[c + 1] = start_gather(c + 1)
        g[c].wait()
        if c >= 2:
            s[c - 2].wait()
        transpose(c)
        s[c] = start_store(c)
    s[total - 2].wait()
    s[total - 1].wait()


@jax.jit
def _pair(i0, i1, t0, t1):
    mesh = plsc.VectorSubcoreMesh(core_axis_name="c", subcore_axis_name="s")
    out = jax.ShapeDtypeStruct((GROUPS * 2, NTILE, 8, CHUNK), jnp.float32)
    return pl.kernel(
        _pair_body,
        out_type=(out, out),
        mesh=mesh,
        compiler_params=pltpu.CompilerParams(use_tc_tiling_on_sc=False,
                                             needs_layout_passes=False),
        scratch_types=[
            pltpu.VMEM((2, B_PER_W), jnp.int32),
            pltpu.VMEM((2, CHUNK, DIM), jnp.float32),
            pltpu.VMEM((2, GROUPS * 2, 8, CHUNK), jnp.float32),
            pltpu.SemaphoreType.DMA((2,)),
            pltpu.SemaphoreType.DMA((2,)),
        ],
    )(i0, i1, t0, t1)


PADDIM = 128


def _pairL_body(i0, i1, t0, t1, o0, o1, idx_v, rows_v, gsem, ssem):
    wid = lax.axis_index("s") * NUM_CORES + lax.axis_index("c")
    base = wid * B_PER_W

    pltpu.sync_copy(i0.at[pl.ds(base, B_PER_W)], idx_v.at[0])
    pltpu.sync_copy(i1.at[pl.ds(base, B_PER_W)], idx_v.at[1])

    g0 = pltpu.async_copy(t0.at[idx_v.at[0]], rows_v.at[0], gsem.at[0])
    g1 = pltpu.async_copy(t1.at[idx_v.at[1]], rows_v.at[1], gsem.at[1])

    dst = pl.ds(base, B_PER_W), pl.ds(0, DIM)
    g0.wait()
    s0 = pltpu.async_copy(rows_v.at[0], o0.at[dst], ssem.at[0])
    g1.wait()
    s1 = pltpu.async_copy(rows_v.at[1], o1.at[dst], ssem.at[1])
    s0.wait()
    s1.wait()


@jax.jit
def _pairL(i0, i1, t0, t1):
    mesh = plsc.VectorSubcoreMesh(core_axis_name="c", subcore_axis_name="s")
    out = jax.ShapeDtypeStruct((BATCH, PADDIM), jnp.float32)
    return pl.kernel(
        _pairL_body,
        out_type=(out, out),
        mesh=mesh,
        compiler_params=pltpu.CompilerParams(use_tc_tiling_on_sc=False),
        scratch_types=[
            pltpu.VMEM((2, B_PER_W), jnp.int32),
            pltpu.VMEM((2, B_PER_W, DIM), jnp.float32),
            pltpu.SemaphoreType.DMA((2,)),
            pltpu.SemaphoreType.DMA((2,)),
        ],
    )(i0, i1, t0, t1)


def _untranspose(o4):
    return o4.transpose(1, 3, 0, 2).reshape(BATCH, DIM)


def kernel(head, relation, tail, timestamp,
           entity_table, relation_table, timestamp_table):
    rel_o, ts_o = _pair(relation.astype(jnp.int32),
                        timestamp.astype(jnp.int32),
                        relation_table, timestamp_table)
    head_o, tail_o = _pairL(head.astype(jnp.int32),
                            tail.astype(jnp.int32),
                            entity_table, entity_table)
    return (head_o[:, :DIM], _untranspose(rel_o),
            tail_o[:, :DIM], _untranspose(ts_o))


# speedup vs baseline: 1.2664x; 1.1978x over previous
import functools

import jax
import jax.numpy as jnp
from jax import lax
from jax.experimental import pallas as pl
from jax.experimental.pallas import tpu as pltpu
from jax.experimental.pallas import tpu_sc as plsc

NUM_CORES = 2
NUM_SUBCORES = 16
NUM_WORKERS = NUM_CORES * NUM_SUBCORES

BATCH = 16384
DIM = 64
PADDIM = 128

B_PER_W = BATCH // NUM_WORKERS


def _pair_body(i0, i1, t0, t1, o0, o1, idx_v, rows_v, gsem, ssem):
    wid = lax.axis_index("s") * NUM_CORES + lax.axis_index("c")
    base = wid * B_PER_W

    pltpu.sync_copy(i0.at[pl.ds(base, B_PER_W)], idx_v.at[0])
    pltpu.sync_copy(i1.at[pl.ds(base, B_PER_W)], idx_v.at[1])

    g0 = pltpu.async_copy(t0.at[idx_v.at[0]], rows_v.at[0], gsem.at[0])
    g1 = pltpu.async_copy(t1.at[idx_v.at[1]], rows_v.at[1], gsem.at[1])

    dst = pl.ds(base, B_PER_W), pl.ds(0, DIM)
    g0.wait()
    s0 = pltpu.async_copy(rows_v.at[0], o0.at[dst], ssem.at[0])
    g1.wait()
    s1 = pltpu.async_copy(rows_v.at[1], o1.at[dst], ssem.at[1])
    s0.wait()
    s1.wait()


@jax.jit
def _pair(i0, i1, t0, t1):
    mesh = plsc.VectorSubcoreMesh(core_axis_name="c", subcore_axis_name="s")
    out = jax.ShapeDtypeStruct((BATCH, PADDIM), jnp.float32)
    return pl.kernel(
        _pair_body,
        out_type=(out, out),
        mesh=mesh,
        compiler_params=pltpu.CompilerParams(use_tc_tiling_on_sc=False),
        scratch_types=[
            pltpu.VMEM((2, B_PER_W), jnp.int32),
            pltpu.VMEM((2, B_PER_W, DIM), jnp.float32),
            pltpu.SemaphoreType.DMA((2,)),
            pltpu.SemaphoreType.DMA((2,)),
        ],
    )(i0, i1, t0, t1)


def kernel(head, relation, tail, timestamp,
           entity_table, relation_table, timestamp_table):
    rel_o, ts_o = _pair(relation.astype(jnp.int32),
                        timestamp.astype(jnp.int32),
                        relation_table, timestamp_table)
    head_o, tail_o = _pair(head.astype(jnp.int32),
                           tail.astype(jnp.int32),
                           entity_table, entity_table)
    return (head_o[:, :DIM], rel_o[:, :DIM],
            tail_o[:, :DIM], ts_o[:, :DIM])


# speedup vs baseline: 1.2733x; 1.0055x over previous
import functools

import jax
import jax.numpy as jnp
from jax import lax
from jax.experimental import pallas as pl
from jax.experimental.pallas import tpu as pltpu
from jax.experimental.pallas import tpu_sc as plsc

NUM_CORES = 2
NUM_SUBCORES = 16
NUM_WORKERS = NUM_CORES * NUM_SUBCORES

BATCH = 16384
DIM = 64
PADDIM = 128

B_PER_W = BATCH // NUM_WORKERS


def _pair_body(i0, i1, t0, t1, o0, o1, idx_v, rows_v, isem, gsem, ssem):
    wid = lax.axis_index("s") * NUM_CORES + lax.axis_index("c")
    base = wid * B_PER_W

    ia = pltpu.async_copy(i0.at[pl.ds(base, B_PER_W)], idx_v.at[0],
                          isem.at[0])
    ib = pltpu.async_copy(i1.at[pl.ds(base, B_PER_W)], idx_v.at[1],
                          isem.at[1])

    ia.wait()
    g0 = pltpu.async_copy(t0.at[idx_v.at[0]], rows_v.at[0], gsem.at[0])
    ib.wait()
    g1 = pltpu.async_copy(t1.at[idx_v.at[1]], rows_v.at[1], gsem.at[1])

    dst = pl.ds(base, B_PER_W), pl.ds(0, DIM)
    g0.wait()
    s0 = pltpu.async_copy(rows_v.at[0], o0.at[dst], ssem.at[0])
    g1.wait()
    s1 = pltpu.async_copy(rows_v.at[1], o1.at[dst], ssem.at[1])
    s0.wait()
    s1.wait()


@jax.jit
def _pair(i0, i1, t0, t1):
    mesh = plsc.VectorSubcoreMesh(core_axis_name="c", subcore_axis_name="s")
    out = jax.ShapeDtypeStruct((BATCH, PADDIM), jnp.float32)
    return pl.kernel(
        _pair_body,
        out_type=(out, out),
        mesh=mesh,
        compiler_params=pltpu.CompilerParams(use_tc_tiling_on_sc=False),
        scratch_types=[
            pltpu.VMEM((2, B_PER_W), jnp.int32),
            pltpu.VMEM((2, B_PER_W, DIM), jnp.float32),
            pltpu.SemaphoreType.DMA((2,)),
            pltpu.SemaphoreType.DMA((2,)),
            pltpu.SemaphoreType.DMA((2,)),
        ],
    )(i0, i1, t0, t1)


def kernel(head, relation, tail, timestamp,
           entity_table, relation_table, timestamp_table):
    rel_o, ts_o = _pair(relation.astype(jnp.int32),
                        timestamp.astype(jnp.int32),
                        relation_table, timestamp_table)
    head_o, tail_o = _pair(head.astype(jnp.int32),
                           tail.astype(jnp.int32),
                           entity_table, entity_table)
    return (head_o[:, :DIM], rel_o[:, :DIM],
            tail_o[:, :DIM], ts_o[:, :DIM])


# speedup vs baseline: 1.2787x; 1.0042x over previous
import jax
import jax.numpy as jnp
from jax import lax
from jax.experimental import pallas as pl
from jax.experimental.pallas import tpu as pltpu
from jax.experimental.pallas import tpu_sc as plsc

NUM_CORES = 2
NUM_SUBCORES = 16
NUM_WORKERS = NUM_CORES * NUM_SUBCORES

BATCH = 16384
DIM = 64
PADDIM = 128

B_PER_W = BATCH // NUM_WORKERS


def _pair_body(i0, i1, t0, t1, o0, o1, idx_v, rows_v, isem, gsem, ssem):
    wid = lax.axis_index("s") * NUM_CORES + lax.axis_index("c")
    base = wid * B_PER_W

    ia = pltpu.async_copy(i0.at[pl.ds(base, B_PER_W)], idx_v.at[0],
                          isem.at[0])
    ib = pltpu.async_copy(i1.at[pl.ds(base, B_PER_W)], idx_v.at[1],
                          isem.at[1])

    ia.wait()
    g0 = pltpu.async_copy(t0.at[idx_v.at[0]], rows_v.at[0], gsem.at[0])
    ib.wait()
    g1 = pltpu.async_copy(t1.at[idx_v.at[1]], rows_v.at[1], gsem.at[1])

    dst = pl.ds(base, B_PER_W), pl.ds(0, DIM)
    g0.wait()
    s0 = pltpu.async_copy(rows_v.at[0], o0.at[dst], ssem.at[0])
    g1.wait()
    s1 = pltpu.async_copy(rows_v.at[1], o1.at[dst], ssem.at[1])
    s0.wait()
    s1.wait()


@jax.jit
def _pair(i0, i1, t0, t1):
    mesh = plsc.VectorSubcoreMesh(core_axis_name="c", subcore_axis_name="s")
    out = jax.ShapeDtypeStruct((BATCH, PADDIM), jnp.float32)
    return pl.kernel(
        _pair_body,
        out_type=(out, out),
        mesh=mesh,
        compiler_params=pltpu.CompilerParams(use_tc_tiling_on_sc=False),
        scratch_types=[
            pltpu.VMEM((2, B_PER_W), jnp.int32),
            pltpu.VMEM((2, B_PER_W, DIM), jnp.float32),
            pltpu.SemaphoreType.DMA((2,)),
            pltpu.SemaphoreType.DMA((2,)),
            pltpu.SemaphoreType.DMA((2,)),
        ],
    )(i0, i1, t0, t1)


def kernel(head, relation, tail, timestamp,
           entity_table, relation_table, timestamp_table):
    rel_o, ts_o = _pair(relation.astype(jnp.int32),
                        timestamp.astype(jnp.int32),
                        relation_table, timestamp_table)
    head_o, tail_o = _pair(head.astype(jnp.int32),
                           tail.astype(jnp.int32),
                           entity_table, entity_table)
    return (head_o[:, :DIM], rel_o[:, :DIM],
            tail_o[:, :DIM], ts_o[:, :DIM])
